# baseline (speedup 1.0000x reference)
import jax
import jax.numpy as jnp
from jax.experimental import pallas as pl
from jax.experimental.pallas import tpu as pltpu

B = 16
W = 128
N = B * W
D_IN = 128
H3 = 256
OUT = 16

_F32 = jnp.float32
_BF16 = jnp.bfloat16


def _dot(a, b):
    return jnp.dot(a.astype(_BF16), b.astype(_BF16),
                   preferred_element_type=_F32)


def _dot16(a, b):
    return _dot(a, b).astype(_BF16)


def _dotT(a, b, dims):
    return jax.lax.dot_general(a.astype(_BF16), b.astype(_BF16),
                               (dims, ((), ())),
                               preferred_element_type=_F32)


def _dotT32(a, b, dims):
    return jax.lax.dot_general(a, b, (dims, ((), ())),
                               preferred_element_type=_F32)


def _fwd_kernel(H_hbm, De_hbm, x_hbm, b_ref, wl_ref,
                W1_ref, b1_ref, W2_ref, b2_ref, W3_ref, b3_ref,
                Wp1_ref, bp1_ref, Wp2_ref, bp2_ref,
                L_hbm, emb_hbm, logp_ref,
                Hv, Dev, xv, Lv, embv, an_ref,
                semH, semDe, semX, semL, semE):
    R = range(B)
    cH = pltpu.make_async_copy(H_hbm, Hv, semH)
    cDe = pltpu.make_async_copy(De_hbm, Dev, semDe)
    cX = pltpu.make_async_copy(x_hbm, xv, semX)
    cH.start()
    cDe.start()
    cX.start()

    wl = jnp.abs(wl_ref[0, :])
    eye = (jax.lax.broadcasted_iota(jnp.int32, (W, W), 0) ==
           jax.lax.broadcasted_iota(jnp.int32, (W, W), 1)).astype(_F32)
    ones_col = jnp.ones((W, 1), dtype=_F32)

    cH.wait()
    Hb = [Hv[k] for k in R]
    Hw = [Hb[k] * wl[None, :] for k in R]
    dinv = [jax.lax.rsqrt(jnp.sum(Hw[k], axis=1, keepdims=True))
            for k in R]
    Hs = [Hb[k] * dinv[k] for k in R]
    cDe.wait()
    M1 = [_dotT(Dev[k], Hs[k], (((1,), (1,)))) for k in R]
    Lb = [_dot(Hw[k], M1[k]) * dinv[k] for k in R]
    for k in R:
        Lv[k] = Lb[k]
    cL = pltpu.make_async_copy(Lv, L_hbm, semL)
    cL.start()

    deg = [_dotT32(Lb[k], ones_col, (((0,), (0,)))) for k in R]
    dis = [jnp.where(deg[k] > 0, jax.lax.rsqrt(deg[k]), 0.0) for k in R]
    dis_l = [_dotT32(dis[k], eye, (((0,), (0,)))) for k in R]
    for k in R:
        an_ref[k] = (Lb[k] * dis[k] * dis_l[k]).astype(_BF16)

    def agg(p, k, bias_ref, out_t):
        out = jax.lax.dot_general(
            an_ref[k], p.astype(_BF16), ((((0,), (0,))), ((), ())),
            preferred_element_type=_F32)
        if out_t == _BF16:
            out = out.astype(_BF16)
        return out + bias_ref[0, :][None, :].astype(out_t)

    cX.wait()
    p1 = [_dot16(xv[k], W1_ref[...]) for k in R]
    h1 = [agg(p1[k], k, b1_ref, _BF16) for k in R]
    p2 = [_dot16(jax.nn.relu(h1[k]), W2_ref[...]) for k in R]
    h2 = [agg(p2[k], k, b2_ref, _BF16) for k in R]
    p3 = [_dot16(jax.nn.relu(h2[k]), W3_ref[...]) for k in R]
    h3 = [agg(p3[k], k, b3_ref, _F32) for k in R]
    for k in R:
        embv[k] = h3[k]
    cE = pltpu.make_async_copy(embv, emb_hbm, semE)
    cE.start()

    seg_ids = jax.lax.broadcasted_iota(jnp.int32, (B, W), 0)
    ohT = [(b_ref[0, k * W:(k + 1) * W][None, :] == seg_ids).astype(_F32)
           for k in R]
    sums = sum(_dot(ohT[k], jax.nn.relu(h3[k])) for k in R)
    cnt = sum(jnp.sum(ohT[k], axis=1, keepdims=True) for k in R)

    pooled = sums / jnp.maximum(cnt, 1.0)
    o = jnp.dot(pooled, Wp1_ref[...],
                preferred_element_type=_F32) + bp1_ref[0, :][None, :]
    o = jnp.dot(o, Wp2_ref[...],
                preferred_element_type=_F32) + bp2_ref[0, :][None, :]
    m = jnp.max(o, axis=1, keepdims=True)
    lse = jnp.log(jnp.sum(jnp.exp(o - m), axis=1, keepdims=True)) + m
    logp_ref[...] = o - lse

    cL.wait()
    cE.wait()


def kernel(x, H, De, batch, y, weight_lap, W1, b1, W2, b2, W3, b3,
           Wp1, bp1, Wp2, bp2):
    del y
    batch2 = batch.astype(jnp.int32).reshape(1, N)
    wl2 = weight_lap.reshape(1, W)
    x3 = x.reshape(B, W, D_IN)

    def v(shape):
        return pl.BlockSpec(shape, lambda: (0,) * len(shape))

    any_spec = pl.BlockSpec(memory_space=pl.ANY)

    in_specs = [
            any_spec,
            any_spec,
            any_spec,
            v((1, N)),
            v((1, W)),
            v((D_IN, H3)), v((1, H3)),
            v((H3, H3)), v((1, H3)),
            v((H3, H3)), v((1, H3)),
            v((H3, H3)), v((1, H3)),
            v((H3, OUT)), v((1, OUT)),
    ]
    out_specs = [
        any_spec,
        any_spec,
        pl.BlockSpec((B, OUT), lambda: (0, 0)),
    ]

    L, emb, logp = pl.pallas_call(
        _fwd_kernel,
        in_specs=in_specs,
        out_specs=out_specs,
        out_shape=[
            jax.ShapeDtypeStruct((B, W, W), _F32),
            jax.ShapeDtypeStruct((B, W, H3), _F32),
            jax.ShapeDtypeStruct((B, OUT), _F32),
        ],
        scratch_shapes=[
            pltpu.VMEM((B, W, W), _F32),
            pltpu.VMEM((B, W, W), _F32),
            pltpu.VMEM((B, W, D_IN), _F32),
            pltpu.VMEM((B, W, W), _F32),
            pltpu.VMEM((B, W, H3), _F32),
            pltpu.VMEM((B, W, W), _BF16),
            pltpu.SemaphoreType.DMA,
            pltpu.SemaphoreType.DMA,
            pltpu.SemaphoreType.DMA,
            pltpu.SemaphoreType.DMA,
            pltpu.SemaphoreType.DMA,
        ],
    )(H, De, x3, batch2, wl2,
      W1, b1.reshape(1, H3), W2, b2.reshape(1, H3), W3, b3.reshape(1, H3),
      Wp1, bp1.reshape(1, H3), Wp2, bp2.reshape(1, OUT))

    return (emb.reshape(N, H3), logp, weight_lap, L)


# speedup vs baseline: 1.2081x; 1.2081x over previous
import jax
import jax.numpy as jnp
from jax.experimental import pallas as pl
from jax.experimental.pallas import tpu as pltpu

B = 16
W = 128
N = B * W
D_IN = 128
H3 = 256
OUT = 16

_F32 = jnp.float32
_BF16 = jnp.bfloat16


def _dot(a, b):
    return jnp.dot(a.astype(_BF16), b.astype(_BF16),
                   preferred_element_type=_F32)


def _dot16(a, b):
    return _dot(a, b).astype(_BF16)


def _dotT(a, b, dims):
    return jax.lax.dot_general(a.astype(_BF16), b.astype(_BF16),
                               (dims, ((), ())),
                               preferred_element_type=_F32)


def _dotT32(a, b, dims):
    return jax.lax.dot_general(a, b, (dims, ((), ())),
                               preferred_element_type=_F32)


GPB = 16
STEPS = B // GPB


def _fwd_kernel(H_ref, De_ref, x_ref, b_ref, wl_ref,
                W1_ref, b1_ref, W2_ref, b2_ref, W3_ref, b3_ref,
                Wp1_ref, bp1_ref, Wp2_ref, bp2_ref,
                L_ref, emb_ref, logp_ref,
                sums_ref, cnt_ref, an_ref):
    g = pl.program_id(0)
    R = range(GPB)
    wl = jnp.abs(wl_ref[0, :])
    eye = (jax.lax.broadcasted_iota(jnp.int32, (W, W), 0) ==
           jax.lax.broadcasted_iota(jnp.int32, (W, W), 1)).astype(_F32)
    ones_col = jnp.ones((W, 1), dtype=_F32)

    Hb = [H_ref[k] for k in R]
    Hw = [Hb[k] * wl[None, :] for k in R]
    dinv = [jax.lax.rsqrt(jnp.sum(Hw[k], axis=1, keepdims=True))
            for k in R]
    Hs = [Hb[k] * dinv[k] for k in R]
    M1 = [_dotT(De_ref[k], Hs[k], (((1,), (1,)))) for k in R]
    Lb = [_dot(Hw[k], M1[k]) * dinv[k] for k in R]
    for k in R:
        L_ref[k] = Lb[k]

    deg = [_dotT32(Lb[k], ones_col, (((0,), (0,)))) for k in R]
    dis = [jnp.where(deg[k] > 0, jax.lax.rsqrt(deg[k]), 0.0) for k in R]
    dis_l = [_dotT32(dis[k], eye, (((0,), (0,)))) for k in R]
    for k in R:
        an_ref[k] = (Lb[k] * dis[k] * dis_l[k]).astype(_BF16)

    def agg(p, k, bias_ref, out_t):
        out = jax.lax.dot_general(
            an_ref[k], p.astype(_BF16), ((((0,), (0,))), ((), ())),
            preferred_element_type=_F32)
        if out_t == _BF16:
            out = out.astype(_BF16)
        return out + bias_ref[0, :][None, :].astype(out_t)

    p1 = [_dot16(x_ref[k], W1_ref[...]) for k in R]
    h1 = [agg(p1[k], k, b1_ref, _BF16) for k in R]
    p2 = [_dot16(jax.nn.relu(h1[k]), W2_ref[...]) for k in R]
    h2 = [agg(p2[k], k, b2_ref, _BF16) for k in R]
    p3 = [_dot16(jax.nn.relu(h2[k]), W3_ref[...]) for k in R]
    h3 = [agg(p3[k], k, b3_ref, _F32) for k in R]
    for k in R:
        emb_ref[k] = h3[k]

    @pl.when(g == 0)
    def _init():
        sums_ref[...] = jnp.zeros_like(sums_ref)
        cnt_ref[...] = jnp.zeros_like(cnt_ref)

    seg_ids = jax.lax.broadcasted_iota(jnp.int32, (B, W), 0)
    ohT = [(b_ref[0, 0, k * W:(k + 1) * W][None, :] == seg_ids).astype(_F32)
           for k in R]
    sums_ref[...] += sum(_dot(ohT[k], jax.nn.relu(emb_ref[k])) for k in R)
    cnt_ref[...] += sum(jnp.sum(ohT[k], axis=1, keepdims=True)
                        for k in R)

    @pl.when(g == STEPS - 1)
    def _head():
        pooled = sums_ref[...] / jnp.maximum(cnt_ref[:, 0:1], 1.0)
        o = jnp.dot(pooled, Wp1_ref[...],
                    preferred_element_type=_F32) + bp1_ref[0, :][None, :]
        o = jnp.dot(o, Wp2_ref[...],
                    preferred_element_type=_F32) + bp2_ref[0, :][None, :]
        m = jnp.max(o, axis=1, keepdims=True)
        lse = jnp.log(jnp.sum(jnp.exp(o - m), axis=1, keepdims=True)) + m
        logp_ref[...] = o - lse


def kernel(x, H, De, batch, y, weight_lap, W1, b1, W2, b2, W3, b3,
           Wp1, bp1, Wp2, bp2):
    del y
    batch3 = batch.astype(jnp.int32).reshape(STEPS, 1, GPB * W)
    wl2 = weight_lap.reshape(1, W)
    x3 = x.reshape(B, W, D_IN)

    def c(shape):
        return pl.BlockSpec(shape, lambda g: (0,) * len(shape))

    in_specs = [
            pl.BlockSpec((GPB, W, W), lambda g: (g, 0, 0)),
            pl.BlockSpec((GPB, W, W), lambda g: (g, 0, 0)),
            pl.BlockSpec((GPB, W, D_IN), lambda g: (g, 0, 0)),
            pl.BlockSpec((1, 1, GPB * W), lambda g: (g, 0, 0)),
            c((1, W)),
            c((D_IN, H3)), c((1, H3)),
            c((H3, H3)), c((1, H3)),
            c((H3, H3)), c((1, H3)),
            c((H3, H3)), c((1, H3)),
            c((H3, OUT)), c((1, OUT)),
    ]
    out_specs = [
        pl.BlockSpec((GPB, W, W), lambda g: (g, 0, 0)),
        pl.BlockSpec((GPB, W, H3), lambda g: (g, 0, 0)),
        pl.BlockSpec((B, OUT), lambda g: (0, 0)),
    ]

    L, emb, logp = pl.pallas_call(
        _fwd_kernel,
        grid=(STEPS,),
        in_specs=in_specs,
        out_specs=out_specs,
        out_shape=[
            jax.ShapeDtypeStruct((B, W, W), _F32),
            jax.ShapeDtypeStruct((B, W, H3), _F32),
            jax.ShapeDtypeStruct((B, OUT), _F32),
        ],
        scratch_shapes=[
            pltpu.VMEM((B, H3), _F32),
            pltpu.VMEM((B, 128), _F32),
            pltpu.VMEM((B, W, W), _BF16),
        ],
    )(H, De, x3, batch3, wl2,
      W1, b1.reshape(1, H3), W2, b2.reshape(1, H3), W3, b3.reshape(1, H3),
      Wp1, bp1.reshape(1, H3), Wp2, bp2.reshape(1, OUT))

    return (emb.reshape(N, H3), logp, weight_lap, L)
